# baseline (speedup 1.0000x reference)
import math

import jax
import jax.numpy as jnp
from jax import lax
from jax.experimental import pallas as pl
from jax.experimental.pallas import tpu as pltpu
from jax.experimental.pallas import tpu_sc as plsc

D_MODEL = 64
SCALE = math.sqrt(D_MODEL)

NC = 2
NS = 16
NW = NC * NS

NBUF = 4
LOOK = 2
DRAIN = NBUF - LOOK
GROUP = 8
DT = D_MODEL // 8
CPAD = 128


def _emb_body(xt_hbm, lut_hbm, p_hbm, idx_v, a_v, c_v, gsem, ssem, *,
              seq, bw):
    wid = lax.axis_index("s") * NC + lax.axis_index("c")
    n_groups = seq // GROUP
    col_base = wid * bw

    jvec = jax.lax.broadcasted_iota(jnp.int32, (16,), 0)
    dt_base = jvec // 8
    di_idx = jvec % 8

    def load_idx(k):
        pltpu.sync_copy(
            xt_hbm.at[pl.ds(k * GROUP, GROUP), pl.ds(col_base, bw)],
            idx_v.at[k % 2])

    def fire_gather(slot, r, b):
        pltpu.async_copy(lut_hbm.at[idx_v.at[slot, r]], a_v.at[b], gsem)

    def wait_gather(b):
        pltpu.make_async_copy(lut_hbm.at[idx_v.at[0, 0]], a_v.at[b],
                              gsem).wait()

    def scale_transpose(b):
        @plsc.parallel_loop(0, D_MODEL, 1, unroll=2)
        def _(d):
            d_idx = jvec * 0 + d
            for t in range(bw // 16):
                v = plsc.load_gather(a_v.at[b], [jvec + t * 16, d_idx])
                c_v[b, d // 8, d % 8, pl.ds(t * 16, 16)] = v * SCALE

    def fire_scatter(s, b):
        pltpu.async_copy(c_v.at[b], p_hbm.at[s].at[:, wid], ssem)

    def wait_scatter():
        pltpu.make_async_copy(c_v.at[0], p_hbm.at[0].at[:, 0],
                              ssem).wait()

    load_idx(0)
    for b in range(LOOK):
        fire_gather(0, b, b % NBUF)

    def group_body(g, *, first, last):
        for b in range(GROUP):
            s = g * GROUP + b
            if b == GROUP - LOOK and not last:
                load_idx(g + 1)
            wait_gather(b % NBUF)
            scale_transpose(b % NBUF)
            fire_scatter(s, b % NBUF)
            if not (first and b < DRAIN):
                wait_scatter()
            if not (last and b >= GROUP - LOOK):
                slot = (g + (1 if b >= GROUP - LOOK else 0)) % 2
                fire_gather(slot, (b + LOOK) % GROUP, (b + LOOK) % NBUF)

    group_body(0, first=True, last=False)

    def mid(g, carry):
        group_body(g, first=False, last=False)
        return carry
    lax.fori_loop(1, n_groups - 1, mid, 0)

    group_body(n_groups - 1, first=False, last=True)

    for _ in range(DRAIN):
        wait_scatter()


@jax.jit
def _run(xt, lut):
    seq, batch = xt.shape
    bw = batch // NW
    mesh = plsc.VectorSubcoreMesh(core_axis_name="c", subcore_axis_name="s",
                                  num_cores=NC, num_subcores=NS)

    def body(xt_ref, lut_ref, p_ref, idx_v, a_v, c_v, gsem, ssem):
        _emb_body(xt_ref, lut_ref, p_ref, idx_v, a_v, c_v, gsem, ssem,
                  seq=seq, bw=bw)

    f = pl.kernel(
        body,
        out_type=jax.ShapeDtypeStruct((seq, DT, NW, 8, bw), jnp.float32),
        mesh=mesh,
        scratch_types=[
            pltpu.VMEM((2, GROUP, bw), jnp.int32),
            pltpu.VMEM((NBUF, bw, D_MODEL), jnp.float32),
            pltpu.VMEM((NBUF, DT, 8, CPAD), jnp.float32),
            pltpu.SemaphoreType.DMA,
            pltpu.SemaphoreType.DMA,
        ],
        compiler_params=pltpu.CompilerParams(use_tc_tiling_on_sc=False,
                                             needs_layout_passes=False),
    )
    p = f(xt, lut)
    return lax.transpose(p, (2, 4, 0, 1, 3)).reshape(batch, seq, D_MODEL)


def kernel(x, lut):
    assert x.shape[0] % NW == 0 and x.shape[1] % GROUP == 0
    return _run(x.T, lut)


# speedup vs baseline: 1.3724x; 1.3724x over previous
import math

import jax
import jax.numpy as jnp
from jax import lax
from jax.experimental import pallas as pl
from jax.experimental.pallas import tpu as pltpu
from jax.experimental.pallas import tpu_sc as plsc

D_MODEL = 64
SCALE = math.sqrt(D_MODEL)

NC = 2
NS = 16
NW = NC * NS

NBUF = 8
LOOK = 4
DRAIN = NBUF - LOOK
GROUP = NBUF


def _emb_body(xt_hbm, lut_hbm, out_hbm, idx_v, rows_v, gsem, ssem, *,
              seq, bw):
    wid = lax.axis_index("s") * NC + lax.axis_index("c")
    n_groups = seq // GROUP
    col_base = wid * bw

    def load_idx(k):
        pltpu.sync_copy(
            xt_hbm.at[pl.ds(k * GROUP, GROUP), pl.ds(col_base, bw)],
            idx_v.at[k % 2])

    def fire_gather(slot, r, b):
        pltpu.async_copy(lut_hbm.at[idx_v.at[slot, r]], rows_v.at[b], gsem)

    def wait_gather(b):
        pltpu.make_async_copy(lut_hbm.at[idx_v.at[0, 0]], rows_v.at[b],
                              gsem).wait()

    def scale(b):
        @plsc.parallel_loop(0, bw, 1, unroll=4)
        def _(r):
            for k in range(D_MODEL // 16):
                rows_v[b, r, pl.ds(k * 16, 16)] = (
                    rows_v[b, r, pl.ds(k * 16, 16)] * SCALE)

    def fire_scatter(s, b):
        pltpu.async_copy(rows_v.at[b],
                         out_hbm.at[pl.ds(col_base, bw), s, pl.ds(0, D_MODEL)],
                         ssem)

    def wait_scatter():
        pltpu.make_async_copy(
            rows_v.at[0],
            out_hbm.at[pl.ds(col_base, bw), 0, pl.ds(0, D_MODEL)],
            ssem).wait()

    load_idx(0)
    for b in range(LOOK):
        fire_gather(0, b, b)

    def group_body(g, *, first, last):
        for b in range(GROUP):
            s = g * GROUP + b
            if b == LOOK and not last:
                load_idx(g + 1)
            wait_gather(b)
            scale(b)
            fire_scatter(s, b)
            if not (first and b < DRAIN):
                wait_scatter()
            if not (last and b >= GROUP - LOOK):
                slot = (g + (1 if b >= GROUP - LOOK else 0)) % 2
                fire_gather(slot, (b + LOOK) % GROUP, (b + LOOK) % NBUF)

    group_body(0, first=True, last=False)

    def mid(g, carry):
        group_body(g, first=False, last=False)
        return carry
    lax.fori_loop(1, n_groups - 1, mid, 0)

    group_body(n_groups - 1, first=False, last=True)

    for _ in range(DRAIN):
        wait_scatter()


@jax.jit
def _run(xt, lut):
    seq, batch = xt.shape
    bw = batch // NW
    mesh = plsc.VectorSubcoreMesh(core_axis_name="c", subcore_axis_name="s",
                                  num_cores=NC, num_subcores=NS)

    def body(xt_ref, lut_ref, out_ref, idx_v, rows_v, gsem, ssem):
        _emb_body(xt_ref, lut_ref, out_ref, idx_v, rows_v, gsem, ssem,
                  seq=seq, bw=bw)

    f = pl.kernel(
        body,
        out_type=jax.ShapeDtypeStruct((batch, seq, 2 * D_MODEL), jnp.float32),
        mesh=mesh,
        scratch_types=[
            pltpu.VMEM((2, GROUP, bw), jnp.int32),
            pltpu.VMEM((NBUF, bw, D_MODEL), jnp.float32),
            pltpu.SemaphoreType.DMA,
            pltpu.SemaphoreType.DMA,
        ],
        compiler_params=pltpu.CompilerParams(use_tc_tiling_on_sc=False),
    )
    return f(xt, lut)


def kernel(x, lut):
    assert x.shape[0] % NW == 0 and x.shape[1] % GROUP == 0
    return _run(x.T, lut)[:, :, :D_MODEL]


# speedup vs baseline: 1.3724x; 1.0000x over previous
import math

import jax
import jax.numpy as jnp
from jax import lax
from jax.experimental import pallas as pl
from jax.experimental.pallas import tpu as pltpu
from jax.experimental.pallas import tpu_sc as plsc

D_MODEL = 64
SCALE = math.sqrt(D_MODEL)

NC = 2
NS = 16
NW = NC * NS

NBUF = 8
LOOK = 6
DRAIN = NBUF - LOOK
GROUP = NBUF


def _emb_body(xt_hbm, lut_hbm, out_hbm, idx_v, rows_v, gsem, ssem, *,
              seq, bw):
    wid = lax.axis_index("s") * NC + lax.axis_index("c")
    n_groups = seq // GROUP
    col_base = wid * bw

    def load_idx(k):
        pltpu.sync_copy(
            xt_hbm.at[pl.ds(k * GROUP, GROUP), pl.ds(col_base, bw)],
            idx_v.at[k % 2])

    def fire_gather(slot, r, b):
        pltpu.async_copy(lut_hbm.at[idx_v.at[slot, r]], rows_v.at[b], gsem)

    def wait_gather(b):
        pltpu.make_async_copy(lut_hbm.at[idx_v.at[0, 0]], rows_v.at[b],
                              gsem).wait()

    def scale(b):
        @plsc.parallel_loop(0, bw, 1, unroll=4)
        def _(r):
            for k in range(D_MODEL // 16):
                rows_v[b, r, pl.ds(k * 16, 16)] = (
                    rows_v[b, r, pl.ds(k * 16, 16)] * SCALE)

    def fire_scatter(s, b):
        pltpu.async_copy(rows_v.at[b],
                         out_hbm.at[pl.ds(col_base, bw), s, pl.ds(0, D_MODEL)],
                         ssem)

    def wait_scatter():
        pltpu.make_async_copy(
            rows_v.at[0],
            out_hbm.at[pl.ds(col_base, bw), 0, pl.ds(0, D_MODEL)],
            ssem).wait()

    load_idx(0)
    for b in range(LOOK):
        fire_gather(0, b, b)

    def group_body(g, *, first, last):
        for b in range(GROUP):
            s = g * GROUP + b
            if b == GROUP - LOOK and not last:
                load_idx(g + 1)
            wait_gather(b)
            scale(b)
            fire_scatter(s, b)
            if not (first and b < DRAIN):
                wait_scatter()
            if not (last and b >= GROUP - LOOK):
                slot = (g + (1 if b >= GROUP - LOOK else 0)) % 2
                fire_gather(slot, (b + LOOK) % GROUP, (b + LOOK) % NBUF)

    group_body(0, first=True, last=False)

    def mid(g, carry):
        group_body(g, first=False, last=False)
        return carry
    lax.fori_loop(1, n_groups - 1, mid, 0)

    group_body(n_groups - 1, first=False, last=True)

    for _ in range(DRAIN):
        wait_scatter()


@jax.jit
def _run(xt, lut):
    seq, batch = xt.shape
    bw = batch // NW
    mesh = plsc.VectorSubcoreMesh(core_axis_name="c", subcore_axis_name="s",
                                  num_cores=NC, num_subcores=NS)

    def body(xt_ref, lut_ref, out_ref, idx_v, rows_v, gsem, ssem):
        _emb_body(xt_ref, lut_ref, out_ref, idx_v, rows_v, gsem, ssem,
                  seq=seq, bw=bw)

    f = pl.kernel(
        body,
        out_type=jax.ShapeDtypeStruct((batch, seq, 2 * D_MODEL), jnp.float32),
        mesh=mesh,
        scratch_types=[
            pltpu.VMEM((2, GROUP, bw), jnp.int32),
            pltpu.VMEM((NBUF, bw, D_MODEL), jnp.float32),
            pltpu.SemaphoreType.DMA,
            pltpu.SemaphoreType.DMA,
        ],
        compiler_params=pltpu.CompilerParams(use_tc_tiling_on_sc=False),
    )
    return f(xt, lut)


def kernel(x, lut):
    assert x.shape[0] % NW == 0 and x.shape[1] % GROUP == 0
    return _run(x.T, lut)[:, :, :D_MODEL]


# speedup vs baseline: 1.3746x; 1.0015x over previous
import math

import jax
import jax.numpy as jnp
from jax import lax
from jax.experimental import pallas as pl
from jax.experimental.pallas import tpu as pltpu
from jax.experimental.pallas import tpu_sc as plsc

D_MODEL = 64
SCALE = math.sqrt(D_MODEL)

NC = 2
NS = 16
NW = NC * NS

NBUF = 8
LOOK = 4
DRAIN = NBUF - LOOK
GROUP = NBUF


def _emb_body(xt_hbm, lut_hbm, out_hbm, idx_v, rows_v, gsem, ssem, *,
              seq, bw):
    wid = lax.axis_index("s") * NC + lax.axis_index("c")
    n_groups = seq // GROUP
    col_base = wid * bw

    def load_idx(k):
        pltpu.sync_copy(
            xt_hbm.at[pl.ds(k * GROUP, GROUP), pl.ds(col_base, bw)],
            idx_v.at[k % 2])

    def fire_gather(slot, r, b):
        pltpu.async_copy(lut_hbm.at[idx_v.at[slot, r]], rows_v.at[b], gsem)

    def wait_gather(b):
        pltpu.make_async_copy(lut_hbm.at[idx_v.at[0, 0]], rows_v.at[b],
                              gsem).wait()

    def scale(b):
        @plsc.parallel_loop(0, bw, 1, unroll=4)
        def _(r):
            for k in range(D_MODEL // 16):
                rows_v[b, r, pl.ds(k * 16, 16)] = (
                    rows_v[b, r, pl.ds(k * 16, 16)] * SCALE)

    def fire_scatter(s, b):
        pltpu.async_copy(rows_v.at[b],
                         out_hbm.at[pl.ds(col_base, bw), s, pl.ds(0, D_MODEL)],
                         ssem)

    def wait_scatter():
        pltpu.make_async_copy(
            rows_v.at[0],
            out_hbm.at[pl.ds(col_base, bw), 0, pl.ds(0, D_MODEL)],
            ssem).wait()

    load_idx(0)
    for b in range(LOOK):
        fire_gather(0, b, b)

    def group_body(g, *, first, last):
        for b in range(GROUP):
            s = g * GROUP + b
            if b == GROUP - LOOK and not last:
                load_idx(g + 1)
            wait_gather(b)
            scale(b)
            fire_scatter(s, b)
            if not (first and b < DRAIN):
                wait_scatter()
            if not (last and b >= GROUP - LOOK):
                slot = (g + (1 if b >= GROUP - LOOK else 0)) % 2
                fire_gather(slot, (b + LOOK) % GROUP, (b + LOOK) % NBUF)

    group_body(0, first=True, last=False)

    def mid(g, carry):
        group_body(g, first=False, last=False)
        return carry
    lax.fori_loop(1, n_groups - 1, mid, 0)

    group_body(n_groups - 1, first=False, last=True)

    for _ in range(DRAIN):
        wait_scatter()


@jax.jit
def _run(xt, lut):
    seq, batch = xt.shape
    bw = batch // NW
    mesh = plsc.VectorSubcoreMesh(core_axis_name="c", subcore_axis_name="s",
                                  num_cores=NC, num_subcores=NS)

    def body(xt_ref, lut_ref, out_ref, idx_v, rows_v, gsem, ssem):
        _emb_body(xt_ref, lut_ref, out_ref, idx_v, rows_v, gsem, ssem,
                  seq=seq, bw=bw)

    f = pl.kernel(
        body,
        out_type=jax.ShapeDtypeStruct((batch, seq, 2 * D_MODEL), jnp.float32),
        mesh=mesh,
        scratch_types=[
            pltpu.VMEM((2, GROUP, bw), jnp.int32),
            pltpu.VMEM((NBUF, bw, D_MODEL), jnp.float32),
            pltpu.SemaphoreType.DMA,
            pltpu.SemaphoreType.DMA,
        ],
        compiler_params=pltpu.CompilerParams(use_tc_tiling_on_sc=False),
    )
    return f(xt, lut)


def kernel(x, lut):
    assert x.shape[0] % NW == 0 and x.shape[1] % GROUP == 0
    return _run(x.T, lut)[:, :, :D_MODEL]
